# Initial kernel scaffold; baseline (speedup 1.0000x reference)
import jax
import jax.numpy as jnp
from jax.experimental import pallas as pl


def kernel(x, topk_idx, weights, W, bias):
    raise NotImplementedError("write your pallas kernel here")


# speedup vs baseline: 1.3340x; 1.3340x over previous
import functools

import jax
import jax.numpy as jnp
from jax.experimental import pallas as pl
from jax.experimental.pallas import tpu as pltpu


def _moe_dense_body(idx_ref, wt_ref, x_ref, w_ref, b_ref, o_ref):
    x = x_ref[...]
    n_experts = w_ref.shape[0]
    n_k = idx_ref.shape[1]
    bt = x.shape[0]

    idx = idx_ref[...]
    wt = wt_ref[...]
    eids = jax.lax.broadcasted_iota(jnp.int32, (bt, n_experts), 1)
    c = jnp.zeros((bt, n_experts), jnp.float32)
    for k in range(n_k):
        c = c + jnp.where(idx[:, k:k + 1] == eids, wt[:, k:k + 1], 0.0)

    acc = jnp.zeros((bt, w_ref.shape[2]), jnp.float32)
    for e in range(n_experts):
        y = jax.lax.dot(x, w_ref[e], preferred_element_type=jnp.float32)
        acc = acc + c[:, e:e + 1] * (y + b_ref[e][None, :])
    o_ref[...] = acc


def kernel(x, topk_idx, weights, W, bias):
    B, D = x.shape
    E, _, DOUT = W.shape
    K = topk_idx.shape[1]
    BT = 256
    grid = (B // BT,)

    xb = x.astype(jnp.bfloat16)
    Wb = W.astype(jnp.bfloat16)
    idx = topk_idx.astype(jnp.int32)

    out = pl.pallas_call(
        _moe_dense_body,
        grid=grid,
        in_specs=[
            pl.BlockSpec((BT, K), lambda i: (i, 0)),
            pl.BlockSpec((BT, K), lambda i: (i, 0)),
            pl.BlockSpec((BT, D), lambda i: (i, 0)),
            pl.BlockSpec((E, D, DOUT), lambda i: (0, 0, 0)),
            pl.BlockSpec((E, DOUT), lambda i: (0, 0)),
        ],
        out_specs=pl.BlockSpec((BT, DOUT), lambda i: (i, 0)),
        out_shape=jax.ShapeDtypeStruct((B, DOUT), jnp.float32),
    )(idx, weights, xb, Wb, bias)
    return out


# speedup vs baseline: 1.7595x; 1.3190x over previous
import functools

import jax
import jax.numpy as jnp
from jax.experimental import pallas as pl
from jax.experimental.pallas import tpu as pltpu


def _moe_dense_body(idx_ref, wt_ref, x_ref, w_ref, b_ref, o_ref, wb_ref):
    n_experts = w_ref.shape[0]
    n_k = idx_ref.shape[1]

    @pl.when(pl.program_id(0) == 0)
    def _cast_weights():
        for e in range(n_experts):
            wb_ref[e] = w_ref[e].astype(jnp.bfloat16)

    x = x_ref[...].astype(jnp.bfloat16)
    bt = x.shape[0]

    idx = idx_ref[...]
    wt = wt_ref[...]
    eids = jax.lax.broadcasted_iota(jnp.int32, (bt, n_experts), 1)
    c = jnp.zeros((bt, n_experts), jnp.float32)
    for k in range(n_k):
        c = c + jnp.where(idx[:, k:k + 1] == eids, wt[:, k:k + 1], 0.0)

    acc = jnp.zeros((bt, w_ref.shape[2]), jnp.float32)
    for e in range(n_experts):
        y = jax.lax.dot(x, wb_ref[e], preferred_element_type=jnp.float32)
        acc = acc + c[:, e:e + 1] * (y + b_ref[e][None, :])
    o_ref[...] = acc


def kernel(x, topk_idx, weights, W, bias):
    B, D = x.shape
    E, _, DOUT = W.shape
    K = topk_idx.shape[1]
    BT = 256
    grid = (B // BT,)

    idx = topk_idx.astype(jnp.int32)

    out = pl.pallas_call(
        _moe_dense_body,
        grid=grid,
        in_specs=[
            pl.BlockSpec((BT, K), lambda i: (i, 0)),
            pl.BlockSpec((BT, K), lambda i: (i, 0)),
            pl.BlockSpec((BT, D), lambda i: (i, 0)),
            pl.BlockSpec((E, D, DOUT), lambda i: (0, 0, 0)),
            pl.BlockSpec((E, DOUT), lambda i: (0, 0)),
        ],
        out_specs=pl.BlockSpec((BT, DOUT), lambda i: (i, 0)),
        out_shape=jax.ShapeDtypeStruct((B, DOUT), jnp.float32),
        scratch_shapes=[pltpu.VMEM((E, D, DOUT), jnp.bfloat16)],
    )(idx, weights, x, W, bias)
    return out
